# Initial kernel scaffold; baseline (speedup 1.0000x reference)
import jax
import jax.numpy as jnp
from jax.experimental import pallas as pl


def kernel(entity, edge_index, edge_type, edge_norm, emb, params1, params2):
    raise NotImplementedError("write your pallas kernel here")


# speedup vs baseline: 4.7093x; 4.7093x over previous
import functools

import jax
import jax.numpy as jnp
from jax import lax
from jax.experimental import pallas as pl
from jax.experimental.pallas import tpu as pltpu
from jax.experimental.pallas import tpu_sc as plsc

N = 10000
NP = 10240
E = 160000
D = 100
R_BASE = 50
NB = 8

DP = 128
NC, NS = 2, 16
NW = NC * NS
EP = 163840
EPW = EP // NW
CH = 128
NCH = EPW // CH
ROWS = EP // CH

_mesh = plsc.VectorSubcoreMesh(core_axis_name="c", subcore_axis_name="s")
_sc_params = pltpu.CompilerParams(needs_layout_passes=False)
_f32 = jnp.float32
_i32 = jnp.int32


def _wid():
    return lax.axis_index("s") * NC + lax.axis_index("c")


@functools.partial(
    pl.kernel,
    out_type=(
        jax.ShapeDtypeStruct((ROWS, CH), _f32),
        jax.ShapeDtypeStruct((NC, NP), _f32),
        jax.ShapeDtypeStruct((EP, DP), _f32),
    ),
    mesh=_mesh,
    scratch_types=[
        pltpu.VMEM((NCH, CH), _i32),
        pltpu.VMEM((NCH, CH), _i32),
        pltpu.VMEM((NCH, CH), _i32),
        pltpu.VMEM((NP,), _f32),
        pltpu.VMEM((NP,), _f32),
        pltpu.VMEM((NCH, CH), _f32),
        pltpu.VMEM((CH, DP), _f32),
        pltpu.VMEM_SHARED((NP,), _f32),
        pltpu.SemaphoreType.DMA,
    ],
    compiler_params=_sc_params,
)
def _sc_edge1(src_h, dst_h, typ_h, p_h, q_h, zeros_h, x_h,
              ex_h, den_h, xj_h,
              src2_v, dst2_v, typ2_v, p_v, q_v, ex2_v, rows_v, den_sp, sem):
    c = lax.axis_index("c")
    s = lax.axis_index("s")
    wid = s * NC + c
    rbase = wid * NCH

    pltpu.sync_copy(src_h.at[pl.ds(rbase, NCH)], src2_v)
    pltpu.sync_copy(dst_h.at[pl.ds(rbase, NCH)], dst2_v)
    pltpu.sync_copy(typ_h.at[pl.ds(rbase, NCH)], typ2_v)
    pltpu.sync_copy(p_h, p_v)
    pltpu.sync_copy(q_h, q_v)

    @pl.when(s == 0)
    def _zero():
        pltpu.sync_copy(zeros_h, den_sp)

    def body(j, _):
        for k in range(CH // 16):
            sl = pl.ds(k * 16, 16)
            dstv = dst2_v[j, sl]
            srcv = src2_v[j, sl]
            typv = typ2_v[j, sl]
            pd = plsc.load_gather(p_v, [dstv])
            qs = plsc.load_gather(q_v, [srcv])
            e = pd + qs
            e = jnp.where(e > 0, e, 0.2 * e)
            basev = lax.rem(typv, R_BASE)
            aug = (basev == 42) | (basev == 43)
            exv = jnp.where(aug, jnp.exp(e), 0.0)
            ex2_v[j, sl] = exv
        return 0

    lax.fori_loop(0, NCH, body, 0)
    pltpu.sync_copy(ex2_v, ex_h.at[pl.ds(rbase, NCH)])

    plsc.subcore_barrier()

    def dadd(j, _):
        pltpu.sync_copy(ex2_v.at[j], den_sp.at[dst2_v.at[j]], add=True)
        return 0

    lax.fori_loop(0, NCH, dadd, 0)

    def gat(j, _):
        pltpu.async_copy(x_h.at[src2_v.at[j]], rows_v, sem).wait()
        pltpu.sync_copy(rows_v, xj_h.at[pl.ds(wid * EPW + j * CH, CH)])
        return 0

    lax.fori_loop(0, NCH, gat, 0)

    plsc.subcore_barrier()

    @pl.when(s == 0)
    def _out():
        pltpu.sync_copy(den_sp, den_h.at[c])


@functools.partial(
    pl.kernel,
    out_type=jax.ShapeDtypeStruct((ROWS, CH), _f32),
    mesh=_mesh,
    scratch_types=[
        pltpu.VMEM((NCH, CH), _i32),
        pltpu.VMEM((NP,), _f32),
        pltpu.VMEM((NP,), _f32),
        pltpu.VMEM((NCH, CH), _f32),
    ],
    compiler_params=_sc_params,
)
def _sc_denp(dst_h, den_h, dene_h, dst2_v, den0_v, den1_v, de2_v):
    wid = _wid()
    rbase = wid * NCH
    pltpu.sync_copy(dst_h.at[pl.ds(rbase, NCH)], dst2_v)
    pltpu.sync_copy(den_h.at[0], den0_v)
    pltpu.sync_copy(den_h.at[1], den1_v)

    def body(j, _):
        for k in range(CH // 16):
            sl = pl.ds(k * 16, 16)
            dstv = dst2_v[j, sl]
            d = plsc.load_gather(den0_v, [dstv]) + plsc.load_gather(den1_v, [dstv])
            de2_v[j, sl] = d
        return 0

    lax.fori_loop(0, NCH, body, 0)
    pltpu.sync_copy(de2_v, dene_h.at[pl.ds(rbase, NCH)])


@functools.partial(
    pl.kernel,
    out_type=jax.ShapeDtypeStruct((NC, NP, DP), _f32),
    mesh=_mesh,
    scratch_types=[
        pltpu.VMEM((NCH, CH), _i32),
        pltpu.VMEM((CH, DP), _f32),
        pltpu.VMEM_SHARED((NP, DP), _f32),
    ],
    compiler_params=_sc_params,
)
def _sc_scatter(dst_h, m_h, zeros2_h, aggr_h, dst2_v, rows_v, aggr_sp):
    c = lax.axis_index("c")
    s = lax.axis_index("s")
    wid = s * NC + c
    rbase = wid * NCH
    pltpu.sync_copy(dst_h.at[pl.ds(rbase, NCH)], dst2_v)

    @pl.when(s == 0)
    def _zero():
        pltpu.sync_copy(zeros2_h, aggr_sp)

    plsc.subcore_barrier()

    def body(j, _):
        pltpu.sync_copy(m_h.at[pl.ds(wid * EPW + j * CH, CH)], rows_v)
        pltpu.sync_copy(rows_v, aggr_sp.at[dst2_v.at[j]], add=True)
        return 0

    lax.fori_loop(0, NCH, body, 0)

    plsc.subcore_barrier()

    @pl.when(s == 0)
    def _out():
        pltpu.sync_copy(aggr_sp, aggr_h.at[c])


_BN = 1024


def _tc_pre_body(x_ref, w_ref, ad_ref, as_ref, wx_ref, pq_ref):
    xb = x_ref[...]
    wx = jnp.dot(xb, w_ref[...], preferred_element_type=_f32)
    wx_ref[...] = wx
    p = lax.dot_general(ad_ref[...], wx, (((1,), (1,)), ((), ())),
                        preferred_element_type=_f32)
    q = lax.dot_general(as_ref[...], wx, (((1,), (1,)), ((), ())),
                        preferred_element_type=_f32)
    pq_ref[...] = jnp.concatenate(
        [p, q, jnp.zeros((6, p.shape[1]), _f32)], axis=0)


def _tc_pre(x_p, gw_p, ad_p, as_p):
    grid = (NP // _BN,)
    return pl.pallas_call(
        _tc_pre_body,
        grid=grid,
        in_specs=[
            pl.BlockSpec((_BN, DP), lambda i: (i, 0)),
            pl.BlockSpec((DP, DP), lambda i: (0, 0)),
            pl.BlockSpec((1, DP), lambda i: (0, 0)),
            pl.BlockSpec((1, DP), lambda i: (0, 0)),
        ],
        out_specs=[
            pl.BlockSpec((_BN, DP), lambda i: (i, 0)),
            pl.BlockSpec((8, _BN), lambda i: (0, i)),
        ],
        out_shape=[
            jax.ShapeDtypeStruct((NP, DP), _f32),
            jax.ShapeDtypeStruct((8, NP), _f32),
        ],
    )(x_p, gw_p, ad_p, as_p)


_BE = 1024


def _tc_mid_body(xj_ref, es_ref, att_ref, bcat_ref, gw_ref, lam_ref, m_ref):
    xj = xj_ref[...]
    es = es_ref[...]
    norm = es[:, 0:1]
    typ = es[:, 1:2]
    ex = es[:, 2:3]
    dene = es[:, 3:4]

    iot = lax.broadcasted_iota(_i32, (xj.shape[0], DP), 1).astype(_f32)
    oh = jnp.where(iot == typ, 1.0, 0.0)
    catt = jnp.dot(oh, att_ref[...], preferred_element_type=_f32)

    y8 = jnp.dot(xj, bcat_ref[...], preferred_element_type=_f32)
    msum = jnp.zeros((xj.shape[0], DP), _f32)
    for b in range(NB):
        msum = msum + catt[:, b:b + 1] * y8[:, b * DP:(b + 1) * DP]

    wxj = jnp.dot(xj, gw_ref[...], preferred_element_type=_f32)

    lam = lam_ref[0]
    gamma = 1.0 / (1.0 + jnp.exp(-lam))
    base = typ - jnp.where(typ >= R_BASE, float(R_BASE), 0.0)
    aug = (base == 42.0) | (base == 43.0)
    scale = jnp.where(aug, ex * gamma / (dene + 1e-16), norm)
    m_ref[...] = scale * jnp.where(aug, wxj, msum)


def _tc_mid(xj, es2, att_p, bcat_p, gw_p, lam1):
    grid = (EP // _BE,)
    return pl.pallas_call(
        _tc_mid_body,
        grid=grid,
        in_specs=[
            pl.BlockSpec((_BE, DP), lambda i: (i, 0)),
            pl.BlockSpec((_BE, 8), lambda i: (i, 0)),
            pl.BlockSpec((DP, NB), lambda i: (0, 0)),
            pl.BlockSpec((DP, NB * DP), lambda i: (0, 0)),
            pl.BlockSpec((DP, DP), lambda i: (0, 0)),
            pl.BlockSpec(memory_space=pltpu.SMEM),
        ],
        out_specs=pl.BlockSpec((_BE, DP), lambda i: (i, 0)),
        out_shape=jax.ShapeDtypeStruct((EP, DP), _f32),
    )(xj, es2, att_p, bcat_p, gw_p, lam1)


def _tc_post_body(ag_ref, x_ref, rt_ref, b_ref, o_ref, *, relu):
    ag = ag_ref[...]
    o = ag[0] + ag[1] + jnp.dot(x_ref[...], rt_ref[...],
                                preferred_element_type=_f32) + b_ref[...]
    if relu:
        o = jnp.maximum(o, 0.0)
    o_ref[...] = o


def _tc_post(aggr, x_p, rt_p, b_p, relu):
    grid = (NP // _BN,)
    return pl.pallas_call(
        functools.partial(_tc_post_body, relu=relu),
        grid=grid,
        in_specs=[
            pl.BlockSpec((NC, _BN, DP), lambda i: (0, i, 0)),
            pl.BlockSpec((_BN, DP), lambda i: (i, 0)),
            pl.BlockSpec((DP, DP), lambda i: (0, 0)),
            pl.BlockSpec((1, DP), lambda i: (0, 0)),
        ],
        out_specs=pl.BlockSpec((_BN, DP), lambda i: (i, 0)),
        out_shape=jax.ShapeDtypeStruct((NP, DP), _f32),
    )(aggr, x_p, rt_p, b_p)


def _pad2(a, r, c):
    return jnp.pad(a, ((0, r - a.shape[0]), (0, c - a.shape[1])))


def kernel(entity, edge_index, edge_type, edge_norm, emb, params1, params2):
    x = jnp.take(emb, entity.astype(_i32), axis=0)
    x_p = _pad2(x, NP, DP)

    src = edge_index[0].astype(_i32)
    dst = edge_index[1].astype(_i32)
    typ = edge_type.astype(_i32)
    padn = EP - E
    src2 = jnp.pad(src, (0, padn)).reshape(ROWS, CH)
    dst2 = jnp.pad(dst, (0, padn)).reshape(ROWS, CH)
    typ2 = jnp.pad(typ, (0, padn)).reshape(ROWS, CH)
    normp = jnp.pad(edge_norm.astype(_f32), (0, padn))
    typf = jnp.pad(edge_type.astype(_f32), (0, padn))

    zeros_n = jnp.zeros((NP,), _f32)
    zeros2 = jnp.zeros((NP, DP), _f32)

    for li, p in enumerate((params1, params2)):
        gw_p = _pad2(p['gat_W'].astype(_f32), DP, DP)
        ad_p = jnp.pad(p['gat_a'][:D].astype(_f32), (0, DP - D)).reshape(1, DP)
        as_p = jnp.pad(p['gat_a'][D:].astype(_f32), (0, DP - D)).reshape(1, DP)
        att_p = p['att'].astype(_f32)
        att_p = jnp.pad(att_p, ((0, DP - att_p.shape[0]), (0, 0)))
        bcat_p = jnp.concatenate(
            [_pad2(p['basis'][b].astype(_f32), DP, DP) for b in range(NB)],
            axis=1)
        rt_p = _pad2(p['root'].astype(_f32), DP, DP)
        b_p = jnp.pad(p['bias'].astype(_f32), (0, DP - D)).reshape(1, DP)
        lam1 = p['lambda_aug'].astype(_f32).reshape(1)

        wx, pq = _tc_pre(x_p, gw_p, ad_p, as_p)
        ex2, den, xj = _sc_edge1(src2, dst2, typ2, pq[0], pq[1], zeros_n, x_p)
        dene2 = _sc_denp(dst2, den)
        es2 = jnp.stack(
            [normp, typf, ex2.reshape(EP), dene2.reshape(EP)], axis=1)
        es2 = jnp.pad(es2, ((0, 0), (0, 4)))
        m = _tc_mid(xj, es2, att_p, bcat_p, gw_p, lam1)
        aggr = _sc_scatter(dst2, m, zeros2)
        x_p = _tc_post(aggr, x_p, rt_p, b_p, relu=(li == 0))

    return x_p[:N, :D]


# speedup vs baseline: 5.1120x; 1.0855x over previous
import functools

import jax
import jax.numpy as jnp
from jax import lax
from jax.experimental import pallas as pl
from jax.experimental.pallas import tpu as pltpu
from jax.experimental.pallas import tpu_sc as plsc

N = 10000
NP = 10240
E = 160000
D = 100
R_BASE = 50
NB = 8

DP = 128
NC, NS = 2, 16
NW = NC * NS
EP = 163840
EPW = EP // NW
CH = 128
NCH = EPW // CH
ROWS = EP // CH

_mesh = plsc.VectorSubcoreMesh(core_axis_name="c", subcore_axis_name="s")
_sc_params = pltpu.CompilerParams(needs_layout_passes=False)
_f32 = jnp.float32
_i32 = jnp.int32


def _wid():
    return lax.axis_index("s") * NC + lax.axis_index("c")


@functools.partial(
    pl.kernel,
    out_type=(
        jax.ShapeDtypeStruct((ROWS, CH), _f32),
        jax.ShapeDtypeStruct((NC, NP), _f32),
        jax.ShapeDtypeStruct((EP, DP), _f32),
    ),
    mesh=_mesh,
    scratch_types=[
        pltpu.VMEM((NCH, CH), _i32),
        pltpu.VMEM((NCH, CH), _i32),
        pltpu.VMEM((NCH, CH), _i32),
        pltpu.VMEM((NP,), _f32),
        pltpu.VMEM((NP,), _f32),
        pltpu.VMEM((NCH, CH), _f32),
        pltpu.VMEM((4, CH, DP), _f32),
        pltpu.VMEM_SHARED((NP,), _f32),
        pltpu.SemaphoreType.DMA,
        pltpu.SemaphoreType.DMA,
        pltpu.SemaphoreType.DMA,
        pltpu.SemaphoreType.DMA,
        pltpu.SemaphoreType.DMA,
        pltpu.SemaphoreType.DMA,
        pltpu.SemaphoreType.DMA,
        pltpu.SemaphoreType.DMA,
    ],
    compiler_params=_sc_params,
)
def _sc_edge1(src_h, dst_h, typ_h, p_h, q_h, zeros_h, x_h,
              ex_h, den_h, xj_h,
              src2_v, dst2_v, typ2_v, p_v, q_v, ex2_v, rows_v, den_sp,
              gs0, gs1, gs2, gs3, ws0, ws1, ws2, ws3):
    c = lax.axis_index("c")
    s = lax.axis_index("s")
    wid = s * NC + c
    rbase = wid * NCH

    pltpu.sync_copy(src_h.at[pl.ds(rbase, NCH)], src2_v)
    pltpu.sync_copy(dst_h.at[pl.ds(rbase, NCH)], dst2_v)
    pltpu.sync_copy(typ_h.at[pl.ds(rbase, NCH)], typ2_v)
    pltpu.sync_copy(p_h, p_v)
    pltpu.sync_copy(q_h, q_v)

    @pl.when(s == 0)
    def _zero():
        pltpu.sync_copy(zeros_h, den_sp)

    def body(j, _):
        for k in range(CH // 16):
            sl = pl.ds(k * 16, 16)
            dstv = dst2_v[j, sl]
            srcv = src2_v[j, sl]
            typv = typ2_v[j, sl]
            pd = plsc.load_gather(p_v, [dstv])
            qs = plsc.load_gather(q_v, [srcv])
            e = pd + qs
            e = jnp.where(e > 0, e, 0.2 * e)
            basev = lax.rem(typv, R_BASE)
            aug = (basev == 42) | (basev == 43)
            exv = jnp.where(aug, jnp.exp(e), 0.0)
            ex2_v[j, sl] = exv
        return 0

    lax.fori_loop(0, NCH, body, 0)
    pltpu.sync_copy(ex2_v, ex_h.at[pl.ds(rbase, NCH)])

    plsc.subcore_barrier()

    def dadd(j, _):
        pltpu.sync_copy(ex2_v.at[j], den_sp.at[dst2_v.at[j]], add=True)
        return 0

    lax.fori_loop(0, NCH, dadd, 0)

    gsems = (gs0, gs1, gs2, gs3)
    wsems = (ws0, ws1, ws2, ws3)

    def gat(j4, _):
        for b in range(4):
            cc = j4 * 4 + b

            @pl.when(j4 > 0)
            def _wprev():
                pltpu.make_async_copy(
                    rows_v.at[b], xj_h.at[pl.ds(wid * EPW, CH)],
                    wsems[b]).wait()

            pltpu.async_copy(x_h.at[src2_v.at[cc]], rows_v.at[b], gsems[b])
        for b in range(4):
            cc = j4 * 4 + b
            pltpu.make_async_copy(x_h.at[src2_v.at[cc]], rows_v.at[b],
                                  gsems[b]).wait()
            pltpu.async_copy(rows_v.at[b],
                             xj_h.at[pl.ds(wid * EPW + cc * CH, CH)],
                             wsems[b])
        return 0

    lax.fori_loop(0, NCH // 4, gat, 0)
    for b in range(4):
        pltpu.make_async_copy(rows_v.at[b], xj_h.at[pl.ds(wid * EPW, CH)],
                              wsems[b]).wait()

    plsc.subcore_barrier()

    @pl.when(s == 0)
    def _out():
        pltpu.sync_copy(den_sp, den_h.at[c])


@functools.partial(
    pl.kernel,
    out_type=jax.ShapeDtypeStruct((ROWS, CH), _f32),
    mesh=_mesh,
    scratch_types=[
        pltpu.VMEM((NCH, CH), _i32),
        pltpu.VMEM((NP,), _f32),
        pltpu.VMEM((NP,), _f32),
        pltpu.VMEM((NCH, CH), _f32),
    ],
    compiler_params=_sc_params,
)
def _sc_denp(dst_h, den_h, dene_h, dst2_v, den0_v, den1_v, de2_v):
    wid = _wid()
    rbase = wid * NCH
    pltpu.sync_copy(dst_h.at[pl.ds(rbase, NCH)], dst2_v)
    pltpu.sync_copy(den_h.at[0], den0_v)
    pltpu.sync_copy(den_h.at[1], den1_v)

    def body(j, _):
        for k in range(CH // 16):
            sl = pl.ds(k * 16, 16)
            dstv = dst2_v[j, sl]
            d = plsc.load_gather(den0_v, [dstv]) + plsc.load_gather(den1_v, [dstv])
            de2_v[j, sl] = d
        return 0

    lax.fori_loop(0, NCH, body, 0)
    pltpu.sync_copy(de2_v, dene_h.at[pl.ds(rbase, NCH)])


@functools.partial(
    pl.kernel,
    out_type=jax.ShapeDtypeStruct((NC, NP, DP), _f32),
    mesh=_mesh,
    scratch_types=[
        pltpu.VMEM((NCH, CH), _i32),
        pltpu.VMEM((2, CH, DP), _f32),
        pltpu.VMEM_SHARED((NP, DP), _f32),
        pltpu.SemaphoreType.DMA,
        pltpu.SemaphoreType.DMA,
    ],
    compiler_params=_sc_params,
)
def _sc_scatter(dst_h, m_h, zeros2_h, aggr_h, dst2_v, rows_v, aggr_sp,
                rs0, rs1):
    c = lax.axis_index("c")
    s = lax.axis_index("s")
    wid = s * NC + c
    rbase = wid * NCH
    pltpu.sync_copy(dst_h.at[pl.ds(rbase, NCH)], dst2_v)

    @pl.when(s == 0)
    def _zero():
        pltpu.sync_copy(zeros2_h, aggr_sp)

    plsc.subcore_barrier()

    rsems = (rs0, rs1)
    ebase = wid * EPW
    pltpu.async_copy(m_h.at[pl.ds(ebase, CH)], rows_v.at[0], rsems[0])

    def body(j2, _):
        for b in range(2):
            cc = j2 * 2 + b
            nxt = j2 * 2 + b + 1

            @pl.when(nxt < NCH)
            def _pref():
                pltpu.async_copy(m_h.at[pl.ds(ebase + nxt * CH, CH)],
                                 rows_v.at[1 - b], rsems[1 - b])

            pltpu.make_async_copy(m_h.at[pl.ds(ebase, CH)], rows_v.at[b],
                                  rsems[b]).wait()
            pltpu.sync_copy(rows_v.at[b], aggr_sp.at[dst2_v.at[cc]], add=True)
        return 0

    lax.fori_loop(0, NCH // 2, body, 0)

    plsc.subcore_barrier()

    @pl.when(s == 0)
    def _out():
        pltpu.sync_copy(aggr_sp, aggr_h.at[c])


_BN = 1024


def _tc_pre_body(x_ref, w_ref, ad_ref, as_ref, wx_ref, pq_ref):
    xb = x_ref[...]
    wx = jnp.dot(xb, w_ref[...], preferred_element_type=_f32)
    wx_ref[...] = wx
    p = lax.dot_general(ad_ref[...], wx, (((1,), (1,)), ((), ())),
                        preferred_element_type=_f32)
    q = lax.dot_general(as_ref[...], wx, (((1,), (1,)), ((), ())),
                        preferred_element_type=_f32)
    pq_ref[...] = jnp.concatenate(
        [p, q, jnp.zeros((6, p.shape[1]), _f32)], axis=0)


def _tc_pre(x_p, gw_p, ad_p, as_p):
    grid = (NP // _BN,)
    return pl.pallas_call(
        _tc_pre_body,
        grid=grid,
        in_specs=[
            pl.BlockSpec((_BN, DP), lambda i: (i, 0)),
            pl.BlockSpec((DP, DP), lambda i: (0, 0)),
            pl.BlockSpec((1, DP), lambda i: (0, 0)),
            pl.BlockSpec((1, DP), lambda i: (0, 0)),
        ],
        out_specs=[
            pl.BlockSpec((_BN, DP), lambda i: (i, 0)),
            pl.BlockSpec((8, _BN), lambda i: (0, i)),
        ],
        out_shape=[
            jax.ShapeDtypeStruct((NP, DP), _f32),
            jax.ShapeDtypeStruct((8, NP), _f32),
        ],
    )(x_p, gw_p, ad_p, as_p)


_BE = 1024


def _tc_mid_body(xj_ref, es_ref, att_ref, bcat_ref, gw_ref, lam_ref, m_ref):
    xj = xj_ref[...]
    es = es_ref[...]
    norm = es[:, 0:1]
    typ = es[:, 1:2]
    ex = es[:, 2:3]
    dene = es[:, 3:4]

    iot = lax.broadcasted_iota(_i32, (xj.shape[0], DP), 1).astype(_f32)
    oh = jnp.where(iot == typ, 1.0, 0.0)
    catt = jnp.dot(oh, att_ref[...], preferred_element_type=_f32)

    y8 = jnp.dot(xj, bcat_ref[...], preferred_element_type=_f32)
    msum = jnp.zeros((xj.shape[0], DP), _f32)
    for b in range(NB):
        msum = msum + catt[:, b:b + 1] * y8[:, b * DP:(b + 1) * DP]

    wxj = jnp.dot(xj, gw_ref[...], preferred_element_type=_f32)

    lam = lam_ref[0]
    gamma = 1.0 / (1.0 + jnp.exp(-lam))
    base = typ - jnp.where(typ >= R_BASE, float(R_BASE), 0.0)
    aug = (base == 42.0) | (base == 43.0)
    scale = jnp.where(aug, ex * gamma / (dene + 1e-16), norm)
    m_ref[...] = scale * jnp.where(aug, wxj, msum)


def _tc_mid(xj, es2, att_p, bcat_p, gw_p, lam1):
    grid = (EP // _BE,)
    return pl.pallas_call(
        _tc_mid_body,
        grid=grid,
        in_specs=[
            pl.BlockSpec((_BE, DP), lambda i: (i, 0)),
            pl.BlockSpec((_BE, 8), lambda i: (i, 0)),
            pl.BlockSpec((DP, NB), lambda i: (0, 0)),
            pl.BlockSpec((DP, NB * DP), lambda i: (0, 0)),
            pl.BlockSpec((DP, DP), lambda i: (0, 0)),
            pl.BlockSpec(memory_space=pltpu.SMEM),
        ],
        out_specs=pl.BlockSpec((_BE, DP), lambda i: (i, 0)),
        out_shape=jax.ShapeDtypeStruct((EP, DP), _f32),
    )(xj, es2, att_p, bcat_p, gw_p, lam1)


def _tc_post_body(ag_ref, x_ref, rt_ref, b_ref, o_ref, *, relu):
    ag = ag_ref[...]
    o = ag[0] + ag[1] + jnp.dot(x_ref[...], rt_ref[...],
                                preferred_element_type=_f32) + b_ref[...]
    if relu:
        o = jnp.maximum(o, 0.0)
    o_ref[...] = o


def _tc_post(aggr, x_p, rt_p, b_p, relu):
    grid = (NP // _BN,)
    return pl.pallas_call(
        functools.partial(_tc_post_body, relu=relu),
        grid=grid,
        in_specs=[
            pl.BlockSpec((NC, _BN, DP), lambda i: (0, i, 0)),
            pl.BlockSpec((_BN, DP), lambda i: (i, 0)),
            pl.BlockSpec((DP, DP), lambda i: (0, 0)),
            pl.BlockSpec((1, DP), lambda i: (0, 0)),
        ],
        out_specs=pl.BlockSpec((_BN, DP), lambda i: (i, 0)),
        out_shape=jax.ShapeDtypeStruct((NP, DP), _f32),
    )(aggr, x_p, rt_p, b_p)


def _pad2(a, r, c):
    return jnp.pad(a, ((0, r - a.shape[0]), (0, c - a.shape[1])))


def kernel(entity, edge_index, edge_type, edge_norm, emb, params1, params2):
    x = jnp.take(emb, entity.astype(_i32), axis=0)
    x_p = _pad2(x, NP, DP)

    src = edge_index[0].astype(_i32)
    dst = edge_index[1].astype(_i32)
    typ = edge_type.astype(_i32)
    padn = EP - E
    src2 = jnp.pad(src, (0, padn)).reshape(ROWS, CH)
    dst2 = jnp.pad(dst, (0, padn)).reshape(ROWS, CH)
    typ2 = jnp.pad(typ, (0, padn)).reshape(ROWS, CH)
    normp = jnp.pad(edge_norm.astype(_f32), (0, padn))
    typf = jnp.pad(edge_type.astype(_f32), (0, padn))

    zeros_n = jnp.zeros((NP,), _f32)
    zeros2 = jnp.zeros((NP, DP), _f32)

    for li, p in enumerate((params1, params2)):
        gw_p = _pad2(p['gat_W'].astype(_f32), DP, DP)
        ad_p = jnp.pad(p['gat_a'][:D].astype(_f32), (0, DP - D)).reshape(1, DP)
        as_p = jnp.pad(p['gat_a'][D:].astype(_f32), (0, DP - D)).reshape(1, DP)
        att_p = p['att'].astype(_f32)
        att_p = jnp.pad(att_p, ((0, DP - att_p.shape[0]), (0, 0)))
        bcat_p = jnp.concatenate(
            [_pad2(p['basis'][b].astype(_f32), DP, DP) for b in range(NB)],
            axis=1)
        rt_p = _pad2(p['root'].astype(_f32), DP, DP)
        b_p = jnp.pad(p['bias'].astype(_f32), (0, DP - D)).reshape(1, DP)
        lam1 = p['lambda_aug'].astype(_f32).reshape(1)

        wx, pq = _tc_pre(x_p, gw_p, ad_p, as_p)
        ex2, den, xj = _sc_edge1(src2, dst2, typ2, pq[0], pq[1], zeros_n, x_p)
        dene2 = _sc_denp(dst2, den)
        es2 = jnp.stack(
            [normp, typf, ex2.reshape(EP), dene2.reshape(EP)], axis=1)
        es2 = jnp.pad(es2, ((0, 0), (0, 4)))
        m = _tc_mid(xj, es2, att_p, bcat_p, gw_p, lam1)
        aggr = _sc_scatter(dst2, m, zeros2)
        x_p = _tc_post(aggr, x_p, rt_p, b_p, relu=(li == 0))

    return x_p[:N, :D]


# speedup vs baseline: 5.1881x; 1.0149x over previous
import functools

import jax
import jax.numpy as jnp
from jax import lax
from jax.experimental import pallas as pl
from jax.experimental.pallas import tpu as pltpu
from jax.experimental.pallas import tpu_sc as plsc

N = 10000
NP = 10240
E = 160000
D = 100
R_BASE = 50
NB = 8

DP = 128
NC, NS = 2, 16
NW = NC * NS
EP = 163840
EPW = EP // NW
CH = 128
NCH = EPW // CH
ROWS = EP // CH

_mesh = plsc.VectorSubcoreMesh(core_axis_name="c", subcore_axis_name="s")
_sc_params = pltpu.CompilerParams(needs_layout_passes=False)
_f32 = jnp.float32
_i32 = jnp.int32


def _wid():
    return lax.axis_index("s") * NC + lax.axis_index("c")


@functools.partial(
    pl.kernel,
    out_type=(
        jax.ShapeDtypeStruct((ROWS, CH), _f32),
        jax.ShapeDtypeStruct((NC, NP), _f32),
        jax.ShapeDtypeStruct((EP, DP), _f32),
    ),
    mesh=_mesh,
    scratch_types=[
        pltpu.VMEM((NCH, CH), _i32),
        pltpu.VMEM((NCH, CH), _i32),
        pltpu.VMEM((NCH, CH), _i32),
        pltpu.VMEM((NP,), _f32),
        pltpu.VMEM((NP,), _f32),
        pltpu.VMEM((NCH, CH), _f32),
        pltpu.VMEM((4, CH, DP), _f32),
        pltpu.VMEM_SHARED((NP,), _f32),
        pltpu.SemaphoreType.DMA,
        pltpu.SemaphoreType.DMA,
        pltpu.SemaphoreType.DMA,
        pltpu.SemaphoreType.DMA,
        pltpu.SemaphoreType.DMA,
        pltpu.SemaphoreType.DMA,
        pltpu.SemaphoreType.DMA,
        pltpu.SemaphoreType.DMA,
    ],
    compiler_params=_sc_params,
)
def _sc_edge1(src_h, dst_h, typ_h, p_h, q_h, zeros_h, x_h,
              ex_h, den_h, xj_h,
              src2_v, dst2_v, typ2_v, p_v, q_v, ex2_v, rows_v, den_sp,
              gs0, gs1, gs2, gs3, ws0, ws1, ws2, ws3):
    c = lax.axis_index("c")
    s = lax.axis_index("s")
    wid = s * NC + c
    rbase = wid * NCH

    pltpu.sync_copy(src_h.at[pl.ds(rbase, NCH)], src2_v)
    pltpu.sync_copy(dst_h.at[pl.ds(rbase, NCH)], dst2_v)
    pltpu.sync_copy(typ_h.at[pl.ds(rbase, NCH)], typ2_v)
    pltpu.sync_copy(p_h, p_v)
    pltpu.sync_copy(q_h, q_v)

    @pl.when(s == 0)
    def _zero():
        pltpu.sync_copy(zeros_h, den_sp)

    gsems = (gs0, gs1, gs2, gs3)
    wsems = (ws0, ws1, ws2, ws3)

    def big(j4, _):
        for b in range(4):
            cc = j4 * 4 + b

            @pl.when(j4 > 0)
            def _wprev():
                pltpu.make_async_copy(
                    rows_v.at[b], xj_h.at[pl.ds(wid * EPW, CH)],
                    wsems[b]).wait()

            pltpu.async_copy(x_h.at[src2_v.at[cc]], rows_v.at[b], gsems[b])
        for b in range(4):
            cc = j4 * 4 + b
            for k in range(CH // 16):
                sl = pl.ds(k * 16, 16)
                dstv = dst2_v[cc, sl]
                srcv = src2_v[cc, sl]
                typv = typ2_v[cc, sl]
                pd = plsc.load_gather(p_v, [dstv])
                qs = plsc.load_gather(q_v, [srcv])
                e = pd + qs
                e = jnp.where(e > 0, e, 0.2 * e)
                basev = lax.rem(typv, R_BASE)
                aug = (basev == 42) | (basev == 43)
                exv = jnp.where(aug, jnp.exp(e), 0.0)
                ex2_v[cc, sl] = exv
        for b in range(4):
            cc = j4 * 4 + b
            pltpu.make_async_copy(x_h.at[src2_v.at[cc]], rows_v.at[b],
                                  gsems[b]).wait()
            pltpu.async_copy(rows_v.at[b],
                             xj_h.at[pl.ds(wid * EPW + cc * CH, CH)],
                             wsems[b])
        return 0

    lax.fori_loop(0, NCH // 4, big, 0)
    for b in range(4):
        pltpu.make_async_copy(rows_v.at[b], xj_h.at[pl.ds(wid * EPW, CH)],
                              wsems[b]).wait()
    pltpu.sync_copy(ex2_v, ex_h.at[pl.ds(rbase, NCH)])

    plsc.subcore_barrier()

    def dadd(j, _):
        pltpu.sync_copy(ex2_v.at[j], den_sp.at[dst2_v.at[j]], add=True)
        return 0

    lax.fori_loop(0, NCH, dadd, 0)

    plsc.subcore_barrier()

    @pl.when(s == 0)
    def _out():
        pltpu.sync_copy(den_sp, den_h.at[c])


@functools.partial(
    pl.kernel,
    out_type=jax.ShapeDtypeStruct((ROWS, CH), _f32),
    mesh=_mesh,
    scratch_types=[
        pltpu.VMEM((NCH, CH), _i32),
        pltpu.VMEM((NP,), _f32),
        pltpu.VMEM((NP,), _f32),
        pltpu.VMEM((NCH, CH), _f32),
    ],
    compiler_params=_sc_params,
)
def _sc_denp(dst_h, den_h, dene_h, dst2_v, den0_v, den1_v, de2_v):
    wid = _wid()
    rbase = wid * NCH
    pltpu.sync_copy(dst_h.at[pl.ds(rbase, NCH)], dst2_v)
    pltpu.sync_copy(den_h.at[0], den0_v)
    pltpu.sync_copy(den_h.at[1], den1_v)

    @plsc.parallel_loop(0, NCH, unroll=4)
    def body(j):
        for k in range(CH // 16):
            sl = pl.ds(k * 16, 16)
            dstv = dst2_v[j, sl]
            d = plsc.load_gather(den0_v, [dstv]) + plsc.load_gather(den1_v, [dstv])
            de2_v[j, sl] = d

    pltpu.sync_copy(de2_v, dene_h.at[pl.ds(rbase, NCH)])


@functools.partial(
    pl.kernel,
    out_type=jax.ShapeDtypeStruct((NC, NP, DP), _f32),
    mesh=_mesh,
    scratch_types=[
        pltpu.VMEM((NCH, CH), _i32),
        pltpu.VMEM((2, CH, DP), _f32),
        pltpu.VMEM_SHARED((NP, DP), _f32),
        pltpu.SemaphoreType.DMA,
        pltpu.SemaphoreType.DMA,
    ],
    compiler_params=_sc_params,
)
def _sc_scatter(dst_h, m_h, zeros2_h, aggr_h, dst2_v, rows_v, aggr_sp,
                rs0, rs1):
    c = lax.axis_index("c")
    s = lax.axis_index("s")
    wid = s * NC + c
    rbase = wid * NCH
    pltpu.sync_copy(dst_h.at[pl.ds(rbase, NCH)], dst2_v)

    @pl.when(s == 0)
    def _zero():
        pltpu.sync_copy(zeros2_h, aggr_sp)

    plsc.subcore_barrier()

    rsems = (rs0, rs1)
    ebase = wid * EPW
    pltpu.async_copy(m_h.at[pl.ds(ebase, CH)], rows_v.at[0], rsems[0])

    def body(j2, _):
        for b in range(2):
            cc = j2 * 2 + b
            nxt = j2 * 2 + b + 1

            @pl.when(nxt < NCH)
            def _pref():
                pltpu.async_copy(m_h.at[pl.ds(ebase + nxt * CH, CH)],
                                 rows_v.at[1 - b], rsems[1 - b])

            pltpu.make_async_copy(m_h.at[pl.ds(ebase, CH)], rows_v.at[b],
                                  rsems[b]).wait()
            pltpu.sync_copy(rows_v.at[b], aggr_sp.at[dst2_v.at[cc]], add=True)
        return 0

    lax.fori_loop(0, NCH // 2, body, 0)

    plsc.subcore_barrier()

    @pl.when(s == 0)
    def _out():
        pltpu.sync_copy(aggr_sp, aggr_h.at[c])


_BN = 1024


def _tc_pre_body(x_ref, w_ref, ad_ref, as_ref, wx_ref, pq_ref):
    xb = x_ref[...]
    wx = jnp.dot(xb, w_ref[...], preferred_element_type=_f32)
    wx_ref[...] = wx
    p = lax.dot_general(ad_ref[...], wx, (((1,), (1,)), ((), ())),
                        preferred_element_type=_f32)
    q = lax.dot_general(as_ref[...], wx, (((1,), (1,)), ((), ())),
                        preferred_element_type=_f32)
    pq_ref[...] = jnp.concatenate(
        [p, q, jnp.zeros((6, p.shape[1]), _f32)], axis=0)


def _tc_pre(x_p, gw_p, ad_p, as_p):
    grid = (NP // _BN,)
    return pl.pallas_call(
        _tc_pre_body,
        grid=grid,
        in_specs=[
            pl.BlockSpec((_BN, DP), lambda i: (i, 0)),
            pl.BlockSpec((DP, DP), lambda i: (0, 0)),
            pl.BlockSpec((1, DP), lambda i: (0, 0)),
            pl.BlockSpec((1, DP), lambda i: (0, 0)),
        ],
        out_specs=[
            pl.BlockSpec((_BN, DP), lambda i: (i, 0)),
            pl.BlockSpec((8, _BN), lambda i: (0, i)),
        ],
        out_shape=[
            jax.ShapeDtypeStruct((NP, DP), _f32),
            jax.ShapeDtypeStruct((8, NP), _f32),
        ],
    )(x_p, gw_p, ad_p, as_p)


_BE = 1024


def _tc_mid_body(xj_ref, es_ref, att_ref, bcat_ref, gw_ref, lam_ref, m_ref):
    xj = xj_ref[...]
    es = es_ref[...]
    norm = es[:, 0:1]
    typ = es[:, 1:2]
    ex = es[:, 2:3]
    dene = es[:, 3:4]

    iot = lax.broadcasted_iota(_i32, (xj.shape[0], DP), 1).astype(_f32)
    oh = jnp.where(iot == typ, 1.0, 0.0)
    catt = jnp.dot(oh, att_ref[...], preferred_element_type=_f32)

    y8 = jnp.dot(xj, bcat_ref[...], preferred_element_type=_f32)
    msum = jnp.zeros((xj.shape[0], DP), _f32)
    for b in range(NB):
        msum = msum + catt[:, b:b + 1] * y8[:, b * DP:(b + 1) * DP]

    wxj = jnp.dot(xj, gw_ref[...], preferred_element_type=_f32)

    lam = lam_ref[0]
    gamma = 1.0 / (1.0 + jnp.exp(-lam))
    base = typ - jnp.where(typ >= R_BASE, float(R_BASE), 0.0)
    aug = (base == 42.0) | (base == 43.0)
    scale = jnp.where(aug, ex * gamma / (dene + 1e-16), norm)
    m_ref[...] = scale * jnp.where(aug, wxj, msum)


def _tc_mid(xj, es2, att_p, bcat_p, gw_p, lam1):
    grid = (EP // _BE,)
    return pl.pallas_call(
        _tc_mid_body,
        grid=grid,
        in_specs=[
            pl.BlockSpec((_BE, DP), lambda i: (i, 0)),
            pl.BlockSpec((_BE, 8), lambda i: (i, 0)),
            pl.BlockSpec((DP, NB), lambda i: (0, 0)),
            pl.BlockSpec((DP, NB * DP), lambda i: (0, 0)),
            pl.BlockSpec((DP, DP), lambda i: (0, 0)),
            pl.BlockSpec(memory_space=pltpu.SMEM),
        ],
        out_specs=pl.BlockSpec((_BE, DP), lambda i: (i, 0)),
        out_shape=jax.ShapeDtypeStruct((EP, DP), _f32),
    )(xj, es2, att_p, bcat_p, gw_p, lam1)


def _tc_post_body(ag_ref, x_ref, rt_ref, b_ref, o_ref, *, relu):
    ag = ag_ref[...]
    o = ag[0] + ag[1] + jnp.dot(x_ref[...], rt_ref[...],
                                preferred_element_type=_f32) + b_ref[...]
    if relu:
        o = jnp.maximum(o, 0.0)
    o_ref[...] = o


def _tc_post(aggr, x_p, rt_p, b_p, relu):
    grid = (NP // _BN,)
    return pl.pallas_call(
        functools.partial(_tc_post_body, relu=relu),
        grid=grid,
        in_specs=[
            pl.BlockSpec((NC, _BN, DP), lambda i: (0, i, 0)),
            pl.BlockSpec((_BN, DP), lambda i: (i, 0)),
            pl.BlockSpec((DP, DP), lambda i: (0, 0)),
            pl.BlockSpec((1, DP), lambda i: (0, 0)),
        ],
        out_specs=pl.BlockSpec((_BN, DP), lambda i: (i, 0)),
        out_shape=jax.ShapeDtypeStruct((NP, DP), _f32),
    )(aggr, x_p, rt_p, b_p)


def _pad2(a, r, c):
    return jnp.pad(a, ((0, r - a.shape[0]), (0, c - a.shape[1])))


def kernel(entity, edge_index, edge_type, edge_norm, emb, params1, params2):
    x = jnp.take(emb, entity.astype(_i32), axis=0)
    x_p = _pad2(x, NP, DP)

    src = edge_index[0].astype(_i32)
    dst = edge_index[1].astype(_i32)
    typ = edge_type.astype(_i32)
    padn = EP - E
    src2 = jnp.pad(src, (0, padn)).reshape(ROWS, CH)
    dst2 = jnp.pad(dst, (0, padn)).reshape(ROWS, CH)
    typ2 = jnp.pad(typ, (0, padn)).reshape(ROWS, CH)
    normp = jnp.pad(edge_norm.astype(_f32), (0, padn))
    typf = jnp.pad(edge_type.astype(_f32), (0, padn))

    zeros_n = jnp.zeros((NP,), _f32)
    zeros2 = jnp.zeros((NP, DP), _f32)

    for li, p in enumerate((params1, params2)):
        gw_p = _pad2(p['gat_W'].astype(_f32), DP, DP)
        ad_p = jnp.pad(p['gat_a'][:D].astype(_f32), (0, DP - D)).reshape(1, DP)
        as_p = jnp.pad(p['gat_a'][D:].astype(_f32), (0, DP - D)).reshape(1, DP)
        att_p = p['att'].astype(_f32)
        att_p = jnp.pad(att_p, ((0, DP - att_p.shape[0]), (0, 0)))
        bcat_p = jnp.concatenate(
            [_pad2(p['basis'][b].astype(_f32), DP, DP) for b in range(NB)],
            axis=1)
        rt_p = _pad2(p['root'].astype(_f32), DP, DP)
        b_p = jnp.pad(p['bias'].astype(_f32), (0, DP - D)).reshape(1, DP)
        lam1 = p['lambda_aug'].astype(_f32).reshape(1)

        wx, pq = _tc_pre(x_p, gw_p, ad_p, as_p)
        ex2, den, xj = _sc_edge1(src2, dst2, typ2, pq[0], pq[1], zeros_n, x_p)
        dene2 = _sc_denp(dst2, den)
        es2 = jnp.stack(
            [normp, typf, ex2.reshape(EP), dene2.reshape(EP)], axis=1)
        es2 = jnp.pad(es2, ((0, 0), (0, 4)))
        m = _tc_mid(xj, es2, att_p, bcat_p, gw_p, lam1)
        aggr = _sc_scatter(dst2, m, zeros2)
        x_p = _tc_post(aggr, x_p, rt_p, b_p, relu=(li == 0))

    return x_p[:N, :D]
